# baseline (speedup 1.0000x reference)
import functools

import jax
import jax.numpy as jnp
from jax import lax
from jax.experimental import pallas as pl
from jax.experimental.pallas import tpu as pltpu
from jax.experimental.pallas import tpu_sc as plsc


def _argmin_body(cbi_ref, cb_ref, z_ref, idx_ref, cbsq_ref, *, bm, bkc, k):
    i = pl.program_id(0)

    @pl.when(i == 0)
    def _():
        c = cb_ref[0]
        cbsq_ref[...] = jnp.sum(c * c, axis=0, keepdims=True)

    zb = z_ref[0]
    zsq = jnp.sum(zb * zb, axis=0, keepdims=True).reshape(bm, 1)
    zn2 = zb * (-2.0)

    run_min = jnp.full((bm, bkc), jnp.inf, jnp.float32)
    run_c = jnp.zeros((bm, bkc), jnp.float32)

    def _dist(c):
        cbc = cb_ref[0, :, c * bkc:(c + 1) * bkc]
        m2 = lax.dot_general(zn2, cbc, (((0,), (0,)), ((), ())),
                             preferred_element_type=jnp.float32)
        return (zsq + m2) + cbsq_ref[0:1, c * bkc:(c + 1) * bkc]

    for c in range(k // bkc):
        dist = _dist(c)
        upd = dist < run_min
        run_min = jnp.minimum(run_min, dist)
        run_c = jnp.where(upd, jnp.float32(c), run_c)
    gmin = jnp.min(run_min, axis=1, keepdims=True)
    lane_f = lax.broadcasted_iota(jnp.int32, (bm, bkc), 1).astype(jnp.float32)
    idx_f = run_c * jnp.float32(bkc) + lane_f
    cand = jnp.where(run_min == gmin, idx_f, jnp.float32(2 ** 24))
    bidx_f = jnp.min(cand, axis=1, keepdims=True)
    idx_ref[...] = bidx_f.astype(jnp.int32)


def _tc_argmin(z_e_x, codebook_index, codebook, b0, nb):
    _, d, t = z_e_x.shape
    k = codebook.shape[-1]
    bm = t
    bkc = 256
    return pl.pallas_call(
        functools.partial(_argmin_body, bm=bm, bkc=bkc, k=k),
        grid_spec=pltpu.PrefetchScalarGridSpec(
            num_scalar_prefetch=1,
            grid=(nb,),
            in_specs=[
                pl.BlockSpec((1, d, k), lambda i, cbi: (cbi[0], 0, 0)),
                pl.BlockSpec((1, d, t), lambda i, cbi: (b0 + i, 0, 0)),
            ],
            out_specs=pl.BlockSpec((bm, 1), lambda i, cbi: (i, 0)),
            scratch_shapes=[pltpu.VMEM((1, k), jnp.float32)],
        ),
        out_shape=jax.ShapeDtypeStruct((nb * t, 1), jnp.int32),
    )(codebook_index, codebook, z_e_x)


def _sc_gather(codebook, codebook_index, ids_flat, b, t):
    _, d, k = codebook.shape
    nw = 32
    dpw = d // nw
    mesh = plsc.VectorSubcoreMesh(core_axis_name="c", subcore_axis_name="s")

    @functools.partial(
        pl.kernel,
        mesh=mesh,
        out_type=jax.ShapeDtypeStruct((b, d, t), jnp.float32),
        compiler_params=pltpu.CompilerParams(needs_layout_passes=False),
        scratch_types=[
            pltpu.VMEM((dpw * k,), jnp.float32),
            pltpu.VMEM((2, t), jnp.int32),
            pltpu.VMEM((2, dpw, t), jnp.float32),
            pltpu.VMEM((16,), jnp.int32),
            pltpu.SemaphoreType.DMA,
            pltpu.SemaphoreType.DMA,
            pltpu.SemaphoreType.DMA,
            pltpu.SemaphoreType.DMA,
        ],
    )
    def run(cb_hbm, cbi_hbm, ids_hbm, out_hbm, cb_v, ids_v, out_v, cbi_v,
            sem_i0, sem_i1, sem_o0, sem_o1):
        wid = lax.axis_index("s") * 2 + lax.axis_index("c")
        pltpu.sync_copy(cbi_hbm, cbi_v.at[pl.ds(0, 1)])
        d0 = wid * dpw
        cbi = cbi_v[pl.ds(0, 16)][0]
        isems = (sem_i0, sem_i1)
        osems = (sem_o0, sem_o1)

        def ids_copy(bi, pb):
            return pltpu.make_async_copy(
                ids_hbm.at[pl.ds(bi * t, t)], ids_v.at[pb], isems[pb])

        def out_copy(bi, pb):
            return pltpu.make_async_copy(
                out_v.at[pb], out_hbm.at[bi, pl.ds(d0, dpw), :], osems[pb])

        ids_copy(0, 0).start()
        for dd in range(dpw):
            pltpu.make_async_copy(
                cb_hbm.at[cbi, d0 + dd, :],
                cb_v.at[pl.ds(dd * k, k)], sem_o0).start()
        for dd in range(dpw):
            pltpu.make_async_copy(
                cb_hbm.at[cbi, d0 + dd, :],
                cb_v.at[pl.ds(dd * k, k)], sem_o0).wait()

        for bi in range(b):
            pb = bi % 2
            ids_copy(bi, pb).wait()
            if bi + 1 < b:
                ids_copy(bi + 1, 1 - pb).start()
            if bi >= 2:
                out_copy(bi - 2, pb).wait()

            def g_body(g, carry2, pb=pb):
                base = g * 16
                idx = ids_v[pb, pl.ds(base, 16)]
                for dd in range(dpw):
                    row = plsc.load_gather(cb_v, [idx + (dd * k)])
                    out_v[pb, dd, pl.ds(base, 16)] = row
                return carry2

            lax.fori_loop(0, t // 16, g_body, 0, unroll=8)
            out_copy(bi, pb).start()
        out_copy(b - 2, 0 if b % 2 == 0 else 1).wait()
        out_copy(b - 1, 1 if b % 2 == 0 else 0).wait()

    return run(codebook, codebook_index, ids_flat)


def kernel(z_e_x, codebook_index, codebook):
    b, d, t = z_e_x.shape
    cbi = codebook_index.astype(jnp.int32)
    h = b // 2
    ids_a = _tc_argmin(z_e_x, cbi, codebook, 0, h)
    zq_a = _sc_gather(codebook, cbi, ids_a.reshape(-1), h, t)
    ids_b = _tc_argmin(z_e_x, cbi, codebook, h, b - h)
    zq_b = _sc_gather(codebook, cbi, ids_b.reshape(-1), b - h, t)
    z_id = jnp.concatenate([ids_a, ids_b], axis=0).reshape(b, t)
    z_q = jnp.concatenate([zq_a, zq_b], axis=0)
    return z_q, z_id


# speedup vs baseline: 1.0089x; 1.0089x over previous
import functools

import jax
import jax.numpy as jnp
from jax import lax
from jax.experimental import pallas as pl
from jax.experimental.pallas import tpu as pltpu
from jax.experimental.pallas import tpu_sc as plsc


def _argmin_body(cbi_ref, cb_ref, z_ref, idx_ref, cbsq_ref, *, bm, bkc, k):
    i = pl.program_id(0)

    @pl.when(i == 0)
    def _():
        c = cb_ref[0]
        cbsq_ref[...] = jnp.sum(c * c, axis=0, keepdims=True)

    zb = z_ref[0]
    zsq = jnp.sum(zb * zb, axis=0, keepdims=True).reshape(bm, 1)
    zn2 = zb * (-2.0)

    run_min = jnp.full((bm, bkc), jnp.inf, jnp.float32)
    run_c = jnp.zeros((bm, bkc), jnp.float32)

    def _dist(c):
        cbc = cb_ref[0, :, c * bkc:(c + 1) * bkc]
        m2 = lax.dot_general(zn2, cbc, (((0,), (0,)), ((), ())),
                             preferred_element_type=jnp.float32)
        return (zsq + m2) + cbsq_ref[0:1, c * bkc:(c + 1) * bkc]

    for c in range(k // bkc):
        dist = _dist(c)
        upd = dist < run_min
        run_min = jnp.minimum(run_min, dist)
        run_c = jnp.where(upd, jnp.float32(c), run_c)
    gmin = jnp.min(run_min, axis=1, keepdims=True)
    lane_f = lax.broadcasted_iota(jnp.int32, (bm, bkc), 1).astype(jnp.float32)
    idx_f = run_c * jnp.float32(bkc) + lane_f
    cand = jnp.where(run_min == gmin, idx_f, jnp.float32(2 ** 24))
    bidx_f = jnp.min(cand, axis=1, keepdims=True)
    idx_ref[...] = bidx_f.astype(jnp.int32)


def _tc_argmin(z_e_x, codebook_index, codebook):
    b, d, t = z_e_x.shape
    k = codebook.shape[-1]
    bm = t
    bkc = 256
    return pl.pallas_call(
        functools.partial(_argmin_body, bm=bm, bkc=bkc, k=k),
        grid_spec=pltpu.PrefetchScalarGridSpec(
            num_scalar_prefetch=1,
            grid=(b,),
            in_specs=[
                pl.BlockSpec((1, d, k), lambda i, cbi: (cbi[0], 0, 0)),
                pl.BlockSpec((1, d, t), lambda i, cbi: (i, 0, 0)),
            ],
            out_specs=pl.BlockSpec((bm, 1), lambda i, cbi: (i, 0)),
            scratch_shapes=[pltpu.VMEM((1, k), jnp.float32)],
        ),
        out_shape=jax.ShapeDtypeStruct((b * t, 1), jnp.int32),
    )(codebook_index, codebook, z_e_x)


def _sc_gather(codebook, codebook_index, ids_flat, b, t):
    _, d, k = codebook.shape
    nw = 32
    dpw = d // nw
    mesh = plsc.VectorSubcoreMesh(core_axis_name="c", subcore_axis_name="s")

    @functools.partial(
        pl.kernel,
        mesh=mesh,
        out_type=jax.ShapeDtypeStruct((b, d, t), jnp.float32),
        compiler_params=pltpu.CompilerParams(needs_layout_passes=False),
        scratch_types=[
            pltpu.VMEM((dpw * k,), jnp.float32),
            pltpu.VMEM((2, t), jnp.int32),
            pltpu.VMEM((2, dpw, t), jnp.float32),
            pltpu.VMEM((16,), jnp.int32),
            pltpu.SemaphoreType.DMA,
            pltpu.SemaphoreType.DMA,
            pltpu.SemaphoreType.DMA,
            pltpu.SemaphoreType.DMA,
        ],
    )
    def run(cb_hbm, cbi_hbm, ids_hbm, out_hbm, cb_v, ids_v, out_v, cbi_v,
            sem_i0, sem_i1, sem_o0, sem_o1):
        wid = lax.axis_index("s") * 2 + lax.axis_index("c")
        pltpu.sync_copy(cbi_hbm, cbi_v.at[pl.ds(0, 1)])
        d0 = wid * dpw
        cbi = cbi_v[pl.ds(0, 16)][0]
        isems = (sem_i0, sem_i1)
        osems = (sem_o0, sem_o1)

        def ids_copy(bi, pb):
            return pltpu.make_async_copy(
                ids_hbm.at[pl.ds(bi * t, t)], ids_v.at[pb], isems[pb])

        def out_copy(bi, pb):
            return pltpu.make_async_copy(
                out_v.at[pb], out_hbm.at[bi, pl.ds(d0, dpw), :], osems[pb])

        ids_copy(0, 0).start()
        for dd in range(dpw):
            pltpu.make_async_copy(
                cb_hbm.at[cbi, d0 + dd, :],
                cb_v.at[pl.ds(dd * k, k)], sem_o0).start()
        for dd in range(dpw):
            pltpu.make_async_copy(
                cb_hbm.at[cbi, d0 + dd, :],
                cb_v.at[pl.ds(dd * k, k)], sem_o0).wait()

        for bi in range(b):
            pb = bi % 2
            ids_copy(bi, pb).wait()
            if bi + 1 < b:
                ids_copy(bi + 1, 1 - pb).start()
            if bi >= 2:
                out_copy(bi - 2, pb).wait()

            def g_body(g, carry2, pb=pb):
                base = g * 16
                idx = ids_v[pb, pl.ds(base, 16)]
                for dd in range(dpw):
                    row = plsc.load_gather(cb_v, [idx + (dd * k)])
                    out_v[pb, dd, pl.ds(base, 16)] = row
                return carry2

            lax.fori_loop(0, t // 16, g_body, 0, unroll=8)
            out_copy(bi, pb).start()
        out_copy(b - 2, 0 if b % 2 == 0 else 1).wait()
        out_copy(b - 1, 1 if b % 2 == 0 else 0).wait()

    return run(codebook, codebook_index, ids_flat)


def kernel(z_e_x, codebook_index, codebook):
    b, d, t = z_e_x.shape
    cbi = codebook_index.astype(jnp.int32)
    ids = _tc_argmin(z_e_x, cbi, codebook)
    z_id = ids.reshape(b, t)
    z_q = _sc_gather(codebook, cbi, ids.reshape(-1), b, t)
    return z_q, z_id
